# baseline (speedup 1.0000x reference)
import functools

import jax
import jax.numpy as jnp
import numpy as np
from jax import lax
from jax.experimental import pallas as pl
from jax.experimental.pallas import tpu as pltpu
from jax.experimental.pallas import tpu_sc as plsc

L = 16
F = 2
T = 524288
N_ROWS = 262144

NC, NS, LANES = 2, 16, 16
NW = NC * NS
PTS_PER_W = N_ROWS // NW
CHUNK = 256
GROUPS = CHUNK // LANES
NCHUNKS = PTS_PER_W // CHUNK
BLK = 8

P1 = int(np.uint32(2654435761).astype(np.int32))
P2 = int(np.uint32(805459861).astype(np.int32))
TM1 = T - 1

CT = 8192
NCT = (T // 2) // CT

_mesh = plsc.VectorSubcoreMesh(
    core_axis_name="c", subcore_axis_name="s", num_cores=NC, num_subcores=NS
)
_cparams = pltpu.CompilerParams(
    needs_layout_passes=False, use_tc_tiling_on_sc=False
)


@functools.partial(
    pl.kernel,
    out_type=jax.ShapeDtypeStruct((L * T,), jnp.int32),
    mesh=_mesh,
    compiler_params=_cparams,
    scratch_types=[
        pltpu.VMEM((CT,), jnp.float32),
        pltpu.VMEM((CT,), jnp.float32),
        pltpu.VMEM((CT,), jnp.int32),
    ],
)
def _interleave_sc(tables_hbm, tflat_hbm, f0_v, f1_v, o_v):
    wid = lax.axis_index("s") * NC + lax.axis_index("c")
    lvl = wid >> 1
    half = wid & 1

    def chunk_body(ci, _):
        t0 = half * (T // 2) + ci * CT
        pltpu.sync_copy(tables_hbm.at[lvl, 0, pl.ds(t0, CT)], f0_v)
        pltpu.sync_copy(tables_hbm.at[lvl, 1, pl.ds(t0, CT)], f1_v)

        def grp(g, _):
            a = plsc.bitcast(f0_v[pl.ds(g * LANES, LANES)], jnp.int32)
            b = plsc.bitcast(f1_v[pl.ds(g * LANES, LANES)], jnp.int32)
            o_v[pl.ds(g * LANES, LANES)] = (
                lax.shift_right_logical(a, 16) | (b & -65536)
            )
            return 0

        lax.fori_loop(0, CT // LANES, grp, 0)
        pltpu.sync_copy(o_v, tflat_hbm.at[pl.ds(lvl * T + t0, CT)])
        return 0

    lax.fori_loop(0, NCT, chunk_body, 0)


_RES = np.floor(16.0 * np.exp(np.log(2048.0 / 16.0) / (L - 1))
                ** np.arange(L)).astype(np.int64)
CACHED = 2
K0 = int(_RES[0]) + 1
K1 = int(_RES[1]) + 1
BC = 2048
BN0 = ((K0 ** 3 + BC - 1) // BC) * BC
BN1 = ((K1 ** 3 + BC - 1) // BC) * BC

TOTAL = (L - CACHED) * NCHUNKS
_CI_BITS = NCHUNKS.bit_length() - 1


def _hashgrid_body(coords_hbm, table_hbm, res_hbm, out_hbm,
                   coords_v, res_v,
                   idx_v0, col_v0, wgt_v0, rows_v0,
                   idx_v1, col_v1, wgt_v1, rows_v1,
                   cache0_v, cache1_v,
                   out_v, sem0, sem1):
    wid = lax.axis_index("s") * NC + lax.axis_index("c")
    base = wid * PTS_PER_W

    pltpu.sync_copy(coords_hbm.at[pl.ds(base, PTS_PER_W), :], coords_v)
    pltpu.sync_copy(res_hbm, res_v)

    iota = lax.iota(jnp.int32, LANES)
    cc0 = jnp.zeros((LANES,), jnp.int32)
    cc1 = jnp.full((LANES,), 1, jnp.int32)
    cc2 = jnp.full((LANES,), 2, jnp.int32)

    def hash_chunk(it, idx_v, col_v, wgt_v):
        lvl = lax.shift_right_logical(it, _CI_BITS) + CACHED
        ci = it & (NCHUNKS - 1)
        resb = res_v[pl.ds(lvl * LANES, LANES)]
        off_l = lvl * T
        pbase = ci * CHUNK

        def grp_hash(g, _):
            prow = pbase + g * LANES + iota
            x = plsc.load_gather(coords_v, [prow, cc0])
            y = plsc.load_gather(coords_v, [prow, cc1])
            z = plsc.load_gather(coords_v, [prow, cc2])
            sx = x * resb
            sy = y * resb
            sz = z * resb
            cx0 = sx.astype(jnp.int32)
            cy0 = sy.astype(jnp.int32)
            cz0 = sz.astype(jnp.int32)
            wx = sx - cx0.astype(jnp.float32)
            wy = sy - cy0.astype(jnp.float32)
            wz = sz - cz0.astype(jnp.float32)
            wxn = 1.0 - wx
            wyn = 1.0 - wy
            wzn = 1.0 - wz
            hx0 = cx0
            hx1 = cx0 + 1
            hy0 = cy0 * P1
            hy1 = hy0 + P1
            hz0 = cz0 * P2
            hz1 = hz0 + P2
            w00 = wxn * wyn
            w01 = wxn * wy
            w10 = wx * wyn
            w11 = wx * wy
            corners = (
                (hx0 ^ hy0 ^ hz0, w00 * wzn),
                (hx0 ^ hy0 ^ hz1, w00 * wz),
                (hx0 ^ hy1 ^ hz0, w01 * wzn),
                (hx0 ^ hy1 ^ hz1, w01 * wz),
                (hx1 ^ hy0 ^ hz0, w10 * wzn),
                (hx1 ^ hy0 ^ hz1, w10 * wz),
                (hx1 ^ hy1 ^ hz0, w11 * wzn),
                (hx1 ^ hy1 ^ hz1, w11 * wz),
            )
            for c, (h, w) in enumerate(corners):
                o = c * CHUNK + g * LANES
                flat = (h & TM1) + off_l
                idx_v[pl.ds(o, LANES)] = lax.shift_right_logical(flat, 3)
                col_v[pl.ds(o, LANES)] = flat & 7
                wgt_v[pl.ds(o, LANES)] = w
            return 0

        lax.fori_loop(0, GROUPS, grp_hash, 0)

    def acc_chunk(it, col_v, wgt_v, rows_v):
        lvl = lax.shift_right_logical(it, _CI_BITS) + CACHED
        ci = it & (NCHUNKS - 1)

        def grp_acc(g, _):
            acc0 = jnp.zeros((LANES,), jnp.float32)
            acc1 = jnp.zeros((LANES,), jnp.float32)
            for c in range(8):
                o = c * CHUNK + g * LANES
                ridx = o + iota
                w = wgt_v[pl.ds(o, LANES)]
                col = col_v[pl.ds(o, LANES)]
                wv = plsc.load_gather(rows_v, [ridx, col])
                f0 = plsc.bitcast(lax.shift_left(wv, 16), jnp.float32)
                f1 = plsc.bitcast(wv & -65536, jnp.float32)
                acc0 = acc0 + f0 * w
                acc1 = acc1 + f1 * w
            out_v[0, pl.ds(g * LANES, LANES)] = acc0
            out_v[1, pl.ds(g * LANES, LANES)] = acc1
            return 0

        lax.fori_loop(0, GROUPS, grp_acc, 0)
        pltpu.sync_copy(
            out_v, out_hbm.at[lvl, :, pl.ds(base + ci * CHUNK, CHUNK)]
        )

    def build_cache(lc, K, BN, cf_v):
        KK = K * K
        off_lc = lc * T

        def bc_body(bc, _):
            def grp_idx(g, _):
                cid = bc * BC + g * LANES + iota
                x = cid // KK
                rem = cid - x * KK
                y = rem // K
                z = rem - y * K
                h = ((x ^ (y * P1) ^ (z * P2)) & TM1) + off_lc
                idx_v0[pl.ds(g * LANES, LANES)] = lax.shift_right_logical(h, 3)
                col_v0[pl.ds(g * LANES, LANES)] = h & 7
                return 0

            lax.fori_loop(0, BC // LANES, grp_idx, 0)
            pltpu.async_copy(table_hbm.at[idx_v0], rows_v0, sem0).wait()

            def grp_fill(g, _):
                jrow = g * LANES + iota
                col = col_v0[pl.ds(g * LANES, LANES)]
                word = plsc.load_gather(rows_v0, [jrow, col])
                cf_v[pl.ds(bc * BC + g * LANES, LANES)] = word
                return 0

            lax.fori_loop(0, BC // LANES, grp_fill, 0)
            return 0

        lax.fori_loop(0, BN // BC, bc_body, 0)

    build_cache(0, K0, BN0, cache0_v)
    build_cache(1, K1, BN1, cache1_v)

    def cached_level(lc, K, cf_v):
        KK = K * K
        resb = res_v[pl.ds(lc * LANES, LANES)]
        kmax = K - 1

        def chunk(ci, _):
            def grp(g, _):
                prow = ci * CHUNK + g * LANES + iota
                x = plsc.load_gather(coords_v, [prow, cc0])
                y = plsc.load_gather(coords_v, [prow, cc1])
                z = plsc.load_gather(coords_v, [prow, cc2])
                sx = x * resb
                sy = y * resb
                sz = z * resb
                cx0 = sx.astype(jnp.int32)
                cy0 = sy.astype(jnp.int32)
                cz0 = sz.astype(jnp.int32)
                wx = sx - cx0.astype(jnp.float32)
                wy = sy - cy0.astype(jnp.float32)
                wz = sz - cz0.astype(jnp.float32)
                wxn = 1.0 - wx
                wyn = 1.0 - wy
                wzn = 1.0 - wz
                xa = jnp.minimum(cx0, kmax) * KK
                xb = jnp.minimum(cx0 + 1, kmax) * KK
                ya = jnp.minimum(cy0, kmax) * K
                yb = jnp.minimum(cy0 + 1, kmax) * K
                za = jnp.minimum(cz0, kmax)
                zb = jnp.minimum(cz0 + 1, kmax)
                w00 = wxn * wyn
                w01 = wxn * wy
                w10 = wx * wyn
                w11 = wx * wy
                corners = (
                    (xa + ya + za, w00 * wzn),
                    (xa + ya + zb, w00 * wz),
                    (xa + yb + za, w01 * wzn),
                    (xa + yb + zb, w01 * wz),
                    (xb + ya + za, w10 * wzn),
                    (xb + ya + zb, w10 * wz),
                    (xb + yb + za, w11 * wzn),
                    (xb + yb + zb, w11 * wz),
                )
                acc0 = jnp.zeros((LANES,), jnp.float32)
                acc1 = jnp.zeros((LANES,), jnp.float32)
                for cell, w in corners:
                    wv = plsc.load_gather(cf_v, [cell])
                    f0 = plsc.bitcast(lax.shift_left(wv, 16), jnp.float32)
                    f1 = plsc.bitcast(wv & -65536, jnp.float32)
                    acc0 = acc0 + f0 * w
                    acc1 = acc1 + f1 * w
                out_v[0, pl.ds(g * LANES, LANES)] = acc0
                out_v[1, pl.ds(g * LANES, LANES)] = acc1
                return 0

            lax.fori_loop(0, GROUPS, grp, 0)
            pltpu.sync_copy(
                out_v, out_hbm.at[lc, :, pl.ds(base + ci * CHUNK, CHUNK)]
            )
            return 0

        lax.fori_loop(0, NCHUNKS, chunk, 0)

    cached_level(0, K0, cache0_v)
    cached_level(1, K1, cache1_v)

    HALF = 4 * CHUNK

    def fire(idx_v, rows_v, sem):
        pltpu.async_copy(
            table_hbm.at[idx_v.at[pl.ds(0, HALF)]],
            rows_v.at[pl.ds(0, HALF), :], sem)
        pltpu.async_copy(
            table_hbm.at[idx_v.at[pl.ds(HALF, HALF)]],
            rows_v.at[pl.ds(HALF, HALF), :], sem)

    def drain(idx_v, rows_v, sem):
        pltpu.make_async_copy(
            table_hbm.at[idx_v.at[pl.ds(0, HALF)]],
            rows_v.at[pl.ds(0, HALF), :], sem).wait()
        pltpu.make_async_copy(
            table_hbm.at[idx_v.at[pl.ds(HALF, HALF)]],
            rows_v.at[pl.ds(HALF, HALF), :], sem).wait()

    hash_chunk(0, idx_v0, col_v0, wgt_v0)
    fire(idx_v0, rows_v0, sem0)
    hash_chunk(1, idx_v1, col_v1, wgt_v1)
    fire(idx_v1, rows_v1, sem1)

    def pair_body(ii, _):
        it = ii * 2
        drain(idx_v0, rows_v0, sem0)
        acc_chunk(it, col_v0, wgt_v0, rows_v0)

        @pl.when(it + 2 < TOTAL)
        def _():
            hash_chunk(it + 2, idx_v0, col_v0, wgt_v0)
            fire(idx_v0, rows_v0, sem0)

        drain(idx_v1, rows_v1, sem1)
        acc_chunk(it + 1, col_v1, wgt_v1, rows_v1)

        @pl.when(it + 3 < TOTAL)
        def _():
            hash_chunk(it + 3, idx_v1, col_v1, wgt_v1)
            fire(idx_v1, rows_v1, sem1)

        return 0

    lax.fori_loop(0, TOTAL // 2, pair_body, 0)


def _build(interpret=False):
    return pl.kernel(
        _hashgrid_body,
        out_type=jax.ShapeDtypeStruct((L, F, N_ROWS), jnp.float32),
        mesh=_mesh,
        compiler_params=_cparams,
        interpret=interpret,
        scratch_types=[
            pltpu.VMEM((PTS_PER_W, 3), jnp.float32),
            pltpu.VMEM((L * LANES,), jnp.float32),
            pltpu.VMEM((8 * CHUNK,), jnp.int32),
            pltpu.VMEM((8 * CHUNK,), jnp.int32),
            pltpu.VMEM((8 * CHUNK,), jnp.float32),
            pltpu.VMEM((8 * CHUNK, BLK), jnp.int32),
            pltpu.VMEM((8 * CHUNK,), jnp.int32),
            pltpu.VMEM((8 * CHUNK,), jnp.int32),
            pltpu.VMEM((8 * CHUNK,), jnp.float32),
            pltpu.VMEM((8 * CHUNK, BLK), jnp.int32),
            pltpu.VMEM((BN0,), jnp.int32),
            pltpu.VMEM((BN1,), jnp.int32),
            pltpu.VMEM((F, CHUNK), jnp.float32),
            pltpu.SemaphoreType.DMA,
            pltpu.SemaphoreType.DMA,
        ],
    )


_hashgrid_sc = _build()


def kernel(coords, tables, resolutions):
    tflat = _interleave_sc(tables)
    table2 = tflat.reshape(L * T // BLK, BLK)
    res_b = jnp.tile(resolutions[:, None], (1, LANES)).reshape(-1)
    return _hashgrid_sc(coords, table2, res_b)


# speedup vs baseline: 1.0064x; 1.0064x over previous
import functools

import jax
import jax.numpy as jnp
import numpy as np
from jax import lax
from jax.experimental import pallas as pl
from jax.experimental.pallas import tpu as pltpu
from jax.experimental.pallas import tpu_sc as plsc

L = 16
F = 2
T = 524288
N_ROWS = 262144

NC, NS, LANES = 2, 16, 16
NW = NC * NS
PTS_PER_W = N_ROWS // NW
CHUNK = 256
GROUPS = CHUNK // LANES
NCHUNKS = PTS_PER_W // CHUNK
BLK = 4

P1 = int(np.uint32(2654435761).astype(np.int32))
P2 = int(np.uint32(805459861).astype(np.int32))
TM1 = T - 1

CT = 8192
NCT = (T // 2) // CT

_mesh = plsc.VectorSubcoreMesh(
    core_axis_name="c", subcore_axis_name="s", num_cores=NC, num_subcores=NS
)
_cparams = pltpu.CompilerParams(
    needs_layout_passes=False, use_tc_tiling_on_sc=False
)


@functools.partial(
    pl.kernel,
    out_type=jax.ShapeDtypeStruct((L * F * T,), jnp.float32),
    mesh=_mesh,
    compiler_params=_cparams,
    scratch_types=[
        pltpu.VMEM((CT,), jnp.float32),
        pltpu.VMEM((CT,), jnp.float32),
        pltpu.VMEM((2 * CT,), jnp.float32),
    ],
)
def _interleave_sc(tables_hbm, tflat_hbm, f0_v, f1_v, o_v):
    wid = lax.axis_index("s") * NC + lax.axis_index("c")
    lvl = wid >> 1
    half = wid & 1
    iota2 = lax.iota(jnp.int32, LANES) * 2

    def chunk_body(ci, _):
        t0 = half * (T // 2) + ci * CT
        pltpu.sync_copy(tables_hbm.at[lvl, 0, pl.ds(t0, CT)], f0_v)
        pltpu.sync_copy(tables_hbm.at[lvl, 1, pl.ds(t0, CT)], f1_v)

        def grp(g, _):
            pos = g * (2 * LANES) + iota2
            a = f0_v[pl.ds(g * LANES, LANES)]
            b = f1_v[pl.ds(g * LANES, LANES)]
            plsc.store_scatter(o_v, [pos], a)
            plsc.store_scatter(o_v, [pos + 1], b)
            return 0

        lax.fori_loop(0, CT // LANES, grp, 0)
        pltpu.sync_copy(o_v, tflat_hbm.at[pl.ds((lvl * T + t0) * F, 2 * CT)])
        return 0

    lax.fori_loop(0, NCT, chunk_body, 0)


_RES = np.floor(16.0 * np.exp(np.log(2048.0 / 16.0) / (L - 1))
                ** np.arange(L)).astype(np.int64)
CACHED = 2
K0 = int(_RES[0]) + 1
K1 = int(_RES[1]) + 1
BC = 2048
BN0 = ((K0 ** 3 + BC - 1) // BC) * BC
BN1 = ((K1 ** 3 + BC - 1) // BC) * BC

TOTAL = (L - CACHED) * NCHUNKS
_CI_BITS = NCHUNKS.bit_length() - 1


def _hashgrid_body(coords_hbm, table_hbm, res_hbm, out_hbm,
                   coords_v, res_v,
                   idx_v0, col_v0, wgt_v0, rows_v0,
                   idx_v1, col_v1, wgt_v1, rows_v1,
                   cache0_v, cache1_v,
                   out_v, sem0, sem1):
    wid = lax.axis_index("s") * NC + lax.axis_index("c")
    base = wid * PTS_PER_W

    pltpu.sync_copy(coords_hbm.at[pl.ds(base, PTS_PER_W), :], coords_v)
    pltpu.sync_copy(res_hbm, res_v)

    iota = lax.iota(jnp.int32, LANES)
    cc0 = jnp.zeros((LANES,), jnp.int32)
    cc1 = jnp.full((LANES,), 1, jnp.int32)
    cc2 = jnp.full((LANES,), 2, jnp.int32)

    def hash_chunk(it, idx_v, col_v, wgt_v):
        lvl = lax.shift_right_logical(it, _CI_BITS) + CACHED
        ci = it & (NCHUNKS - 1)
        resb = res_v[pl.ds(lvl * LANES, LANES)]
        off_l = lvl * T
        pbase = ci * CHUNK

        def grp_hash(g, _):
            prow = pbase + g * LANES + iota
            x = plsc.load_gather(coords_v, [prow, cc0])
            y = plsc.load_gather(coords_v, [prow, cc1])
            z = plsc.load_gather(coords_v, [prow, cc2])
            sx = x * resb
            sy = y * resb
            sz = z * resb
            cx0 = sx.astype(jnp.int32)
            cy0 = sy.astype(jnp.int32)
            cz0 = sz.astype(jnp.int32)
            wx = sx - cx0.astype(jnp.float32)
            wy = sy - cy0.astype(jnp.float32)
            wz = sz - cz0.astype(jnp.float32)
            wxn = 1.0 - wx
            wyn = 1.0 - wy
            wzn = 1.0 - wz
            hx0 = cx0
            hx1 = cx0 + 1
            hy0 = cy0 * P1
            hy1 = hy0 + P1
            hz0 = cz0 * P2
            hz1 = hz0 + P2
            w00 = wxn * wyn
            w01 = wxn * wy
            w10 = wx * wyn
            w11 = wx * wy
            corners = (
                (hx0 ^ hy0 ^ hz0, w00 * wzn),
                (hx0 ^ hy0 ^ hz1, w00 * wz),
                (hx0 ^ hy1 ^ hz0, w01 * wzn),
                (hx0 ^ hy1 ^ hz1, w01 * wz),
                (hx1 ^ hy0 ^ hz0, w10 * wzn),
                (hx1 ^ hy0 ^ hz1, w10 * wz),
                (hx1 ^ hy1 ^ hz0, w11 * wzn),
                (hx1 ^ hy1 ^ hz1, w11 * wz),
            )
            for c, (h, w) in enumerate(corners):
                o = c * CHUNK + g * LANES
                flat = (h & TM1) + off_l
                idx_v[pl.ds(o, LANES)] = lax.shift_right_logical(flat, 2)
                col_v[pl.ds(o, LANES)] = (flat & 3) * F
                wgt_v[pl.ds(o, LANES)] = w
            return 0

        lax.fori_loop(0, GROUPS, grp_hash, 0)

    def acc_chunk(it, col_v, wgt_v, rows_v):
        lvl = lax.shift_right_logical(it, _CI_BITS) + CACHED
        ci = it & (NCHUNKS - 1)

        def grp_acc(g, _):
            acc0 = jnp.zeros((LANES,), jnp.float32)
            acc1 = jnp.zeros((LANES,), jnp.float32)
            for c in range(8):
                o = c * CHUNK + g * LANES
                ridx = o + iota
                w = wgt_v[pl.ds(o, LANES)]
                col = col_v[pl.ds(o, LANES)]
                f0 = plsc.load_gather(rows_v, [ridx, col])
                f1 = plsc.load_gather(rows_v, [ridx, col + 1])
                acc0 = acc0 + f0 * w
                acc1 = acc1 + f1 * w
            out_v[0, pl.ds(g * LANES, LANES)] = acc0
            out_v[1, pl.ds(g * LANES, LANES)] = acc1
            return 0

        lax.fori_loop(0, GROUPS, grp_acc, 0)
        pltpu.sync_copy(
            out_v, out_hbm.at[lvl, :, pl.ds(base + ci * CHUNK, CHUNK)]
        )

    def build_cache(lc, K, BN, cf_v):
        KK = K * K
        off_lc = lc * T

        def bc_body(bc, _):
            def grp_idx(g, _):
                cid = bc * BC + g * LANES + iota
                x = cid // KK
                rem = cid - x * KK
                y = rem // K
                z = rem - y * K
                h = ((x ^ (y * P1) ^ (z * P2)) & TM1) + off_lc
                idx_v0[pl.ds(g * LANES, LANES)] = lax.shift_right_logical(h, 2)
                col_v0[pl.ds(g * LANES, LANES)] = (h & 3) * F
                return 0

            lax.fori_loop(0, BC // LANES, grp_idx, 0)
            pltpu.async_copy(table_hbm.at[idx_v0], rows_v0, sem0).wait()

            def grp_fill(g, _):
                jrow = g * LANES + iota
                col = col_v0[pl.ds(g * LANES, LANES)]
                f0 = plsc.load_gather(rows_v0, [jrow, col])
                f1 = plsc.load_gather(rows_v0, [jrow, col + 1])
                w0 = plsc.bitcast(f0, jnp.int32)
                w1 = plsc.bitcast(f1, jnp.int32)
                word = lax.shift_right_logical(w0, 16) | (w1 & -65536)
                cf_v[pl.ds(bc * BC + g * LANES, LANES)] = word
                return 0

            lax.fori_loop(0, BC // LANES, grp_fill, 0)
            return 0

        lax.fori_loop(0, BN // BC, bc_body, 0)

    build_cache(0, K0, BN0, cache0_v)
    build_cache(1, K1, BN1, cache1_v)

    def cached_level(lc, K, cf_v):
        KK = K * K
        resb = res_v[pl.ds(lc * LANES, LANES)]
        kmax = K - 1

        def chunk(ci, _):
            def grp(g, _):
                prow = ci * CHUNK + g * LANES + iota
                x = plsc.load_gather(coords_v, [prow, cc0])
                y = plsc.load_gather(coords_v, [prow, cc1])
                z = plsc.load_gather(coords_v, [prow, cc2])
                sx = x * resb
                sy = y * resb
                sz = z * resb
                cx0 = sx.astype(jnp.int32)
                cy0 = sy.astype(jnp.int32)
                cz0 = sz.astype(jnp.int32)
                wx = sx - cx0.astype(jnp.float32)
                wy = sy - cy0.astype(jnp.float32)
                wz = sz - cz0.astype(jnp.float32)
                wxn = 1.0 - wx
                wyn = 1.0 - wy
                wzn = 1.0 - wz
                xa = jnp.minimum(cx0, kmax) * KK
                xb = jnp.minimum(cx0 + 1, kmax) * KK
                ya = jnp.minimum(cy0, kmax) * K
                yb = jnp.minimum(cy0 + 1, kmax) * K
                za = jnp.minimum(cz0, kmax)
                zb = jnp.minimum(cz0 + 1, kmax)
                w00 = wxn * wyn
                w01 = wxn * wy
                w10 = wx * wyn
                w11 = wx * wy
                corners = (
                    (xa + ya + za, w00 * wzn),
                    (xa + ya + zb, w00 * wz),
                    (xa + yb + za, w01 * wzn),
                    (xa + yb + zb, w01 * wz),
                    (xb + ya + za, w10 * wzn),
                    (xb + ya + zb, w10 * wz),
                    (xb + yb + za, w11 * wzn),
                    (xb + yb + zb, w11 * wz),
                )
                acc0 = jnp.zeros((LANES,), jnp.float32)
                acc1 = jnp.zeros((LANES,), jnp.float32)
                for cell, w in corners:
                    wv = plsc.load_gather(cf_v, [cell])
                    f0 = plsc.bitcast(lax.shift_left(wv, 16), jnp.float32)
                    f1 = plsc.bitcast(wv & -65536, jnp.float32)
                    acc0 = acc0 + f0 * w
                    acc1 = acc1 + f1 * w
                out_v[0, pl.ds(g * LANES, LANES)] = acc0
                out_v[1, pl.ds(g * LANES, LANES)] = acc1
                return 0

            lax.fori_loop(0, GROUPS, grp, 0)
            pltpu.sync_copy(
                out_v, out_hbm.at[lc, :, pl.ds(base + ci * CHUNK, CHUNK)]
            )
            return 0

        lax.fori_loop(0, NCHUNKS, chunk, 0)

    cached_level(0, K0, cache0_v)
    cached_level(1, K1, cache1_v)

    HALF = 4 * CHUNK

    def fire(idx_v, rows_v, sem):
        pltpu.async_copy(
            table_hbm.at[idx_v.at[pl.ds(0, HALF)]],
            rows_v.at[pl.ds(0, HALF), :], sem)
        pltpu.async_copy(
            table_hbm.at[idx_v.at[pl.ds(HALF, HALF)]],
            rows_v.at[pl.ds(HALF, HALF), :], sem)

    def drain(idx_v, rows_v, sem):
        pltpu.make_async_copy(
            table_hbm.at[idx_v.at[pl.ds(0, HALF)]],
            rows_v.at[pl.ds(0, HALF), :], sem).wait()
        pltpu.make_async_copy(
            table_hbm.at[idx_v.at[pl.ds(HALF, HALF)]],
            rows_v.at[pl.ds(HALF, HALF), :], sem).wait()

    hash_chunk(0, idx_v0, col_v0, wgt_v0)
    fire(idx_v0, rows_v0, sem0)
    hash_chunk(1, idx_v1, col_v1, wgt_v1)
    fire(idx_v1, rows_v1, sem1)

    def pair_body(ii, _):
        it = ii * 2
        drain(idx_v0, rows_v0, sem0)
        acc_chunk(it, col_v0, wgt_v0, rows_v0)

        @pl.when(it + 2 < TOTAL)
        def _():
            hash_chunk(it + 2, idx_v0, col_v0, wgt_v0)
            fire(idx_v0, rows_v0, sem0)

        drain(idx_v1, rows_v1, sem1)
        acc_chunk(it + 1, col_v1, wgt_v1, rows_v1)

        @pl.when(it + 3 < TOTAL)
        def _():
            hash_chunk(it + 3, idx_v1, col_v1, wgt_v1)
            fire(idx_v1, rows_v1, sem1)

        return 0

    lax.fori_loop(0, TOTAL // 2, pair_body, 0)


def _build(interpret=False):
    return pl.kernel(
        _hashgrid_body,
        out_type=jax.ShapeDtypeStruct((L, F, N_ROWS), jnp.float32),
        mesh=_mesh,
        compiler_params=_cparams,
        interpret=interpret,
        scratch_types=[
            pltpu.VMEM((PTS_PER_W, 3), jnp.float32),
            pltpu.VMEM((L * LANES,), jnp.float32),
            pltpu.VMEM((8 * CHUNK,), jnp.int32),
            pltpu.VMEM((8 * CHUNK,), jnp.int32),
            pltpu.VMEM((8 * CHUNK,), jnp.float32),
            pltpu.VMEM((8 * CHUNK, BLK * F), jnp.float32),
            pltpu.VMEM((8 * CHUNK,), jnp.int32),
            pltpu.VMEM((8 * CHUNK,), jnp.int32),
            pltpu.VMEM((8 * CHUNK,), jnp.float32),
            pltpu.VMEM((8 * CHUNK, BLK * F), jnp.float32),
            pltpu.VMEM((BN0,), jnp.int32),
            pltpu.VMEM((BN1,), jnp.int32),
            pltpu.VMEM((F, CHUNK), jnp.float32),
            pltpu.SemaphoreType.DMA,
            pltpu.SemaphoreType.DMA,
        ],
    )


_hashgrid_sc = _build()


def kernel(coords, tables, resolutions):
    tflat = _interleave_sc(tables)
    table2 = tflat.reshape(L * T // BLK, BLK * F)
    res_b = jnp.tile(resolutions[:, None], (1, LANES)).reshape(-1)
    return _hashgrid_sc(coords, table2, res_b)
